# baseline (speedup 1.0000x reference)
import functools

import jax
import jax.numpy as jnp
from jax import lax
from jax.experimental import pallas as pl
from jax.experimental.pallas import tpu as pltpu
from jax.experimental.pallas import tpu_sc as plsc

S = 2048
B = 4
D = 1024
N_TOK = S * B
NC = 2
NS = 16
NW = NC * NS
TOK_PER_W = N_TOK // NW
GCH = 32
NCH = TOK_PER_W // GCH

_MESH = plsc.VectorSubcoreMesh(core_axis_name="c", subcore_axis_name="s")


@functools.partial(
    pl.kernel,
    mesh=_MESH,
    compiler_params=pltpu.CompilerParams(needs_layout_passes=False),
    out_type=jax.ShapeDtypeStruct((N_TOK, D), jnp.float32),
    scratch_types=[
        pltpu.VMEM((NCH, GCH), jnp.int32),
        pltpu.VMEM((2, GCH, D), jnp.float32),
        pltpu.SemaphoreType.DMA,
        pltpu.SemaphoreType.DMA,
        pltpu.SemaphoreType.DMA,
        pltpu.SemaphoreType.DMA,
    ],
)
def _sc_gather(x_hbm, tok_hbm, out_hbm, idx_v, buf_v, g0, g1, o0, o1):
    wid = lax.axis_index("s") * NC + lax.axis_index("c")
    base = wid * TOK_PER_W
    gsem = (g0, g1)
    osem = (o0, o1)

    pltpu.sync_copy(x_hbm.at[wid], idx_v)

    gath = [None, None]
    outc = [None, None]
    for c in range(NCH):
        slot = c & 1
        if outc[slot] is not None:
            outc[slot].wait()
        gath[slot] = pltpu.async_copy(
            tok_hbm.at[idx_v.at[c]], buf_v.at[slot], gsem[slot])
        prev = slot ^ 1
        if gath[prev] is not None:
            gath[prev].wait()
            obase = pl.multiple_of(base + (c - 1) * GCH, GCH)
            outc[prev] = pltpu.async_copy(
                buf_v.at[prev], out_hbm.at[pl.ds(obase, GCH)], osem[prev])
    last = (NCH - 1) & 1
    gath[last].wait()
    obase = pl.multiple_of(base + (NCH - 1) * GCH, GCH)
    outc[last] = pltpu.async_copy(
        buf_v.at[last], out_hbm.at[pl.ds(obase, GCH)], osem[last])
    outc[last ^ 1].wait()
    outc[last].wait()


RS = 128


def _tc_ln_body(h_ref, pos_ref, g_ref, b_ref, o_ref):
    h = h_ref[...] + pos_ref[...]
    mean = jnp.mean(h, axis=-1, keepdims=True)
    cent = h - mean
    var = jnp.mean(cent * cent, axis=-1, keepdims=True)
    o_ref[...] = cent * lax.rsqrt(var + 1e-5) * g_ref[...] + b_ref[...]


_tc_ln = pl.pallas_call(
    _tc_ln_body,
    grid=(S // RS,),
    in_specs=[
        pl.BlockSpec((RS, B, D), lambda i: (i, 0, 0)),
        pl.BlockSpec((RS, 1, D), lambda i: (i, 0, 0)),
        pl.BlockSpec((1, 1, D), lambda i: (0, 0, 0)),
        pl.BlockSpec((1, 1, D), lambda i: (0, 0, 0)),
    ],
    out_specs=pl.BlockSpec((RS, B, D), lambda i: (i, 0, 0)),
    out_shape=jax.ShapeDtypeStruct((S, B, D), jnp.float32),
    compiler_params=pltpu.CompilerParams(
        dimension_semantics=("arbitrary",),
    ),
)


def kernel(x, tok_table, pos_table, gamma, beta):
    xf = x.reshape(NW, NCH, GCH).astype(jnp.int32)
    rows = _sc_gather(xf, tok_table)
    return _tc_ln(
        rows.reshape(S, B, D),
        pos_table.reshape(S, 1, D),
        gamma.reshape(1, 1, D),
        beta.reshape(1, 1, D),
    )


# speedup vs baseline: 1.0313x; 1.0313x over previous
import functools

import jax
import jax.numpy as jnp
from jax import lax
from jax.experimental import pallas as pl
from jax.experimental.pallas import tpu as pltpu
from jax.experimental.pallas import tpu_sc as plsc

S = 2048
B = 4
D = 1024
N_TOK = S * B
NC = 2
NS = 16
NW = NC * NS
HALF = N_TOK // 2
TOK_PER_W = HALF // NW
GCH = 32
NCH = TOK_PER_W // GCH
S_HALF = S // 2

_MESH = plsc.VectorSubcoreMesh(core_axis_name="c", subcore_axis_name="s")


@functools.partial(
    pl.kernel,
    mesh=_MESH,
    compiler_params=pltpu.CompilerParams(needs_layout_passes=False),
    out_type=jax.ShapeDtypeStruct((HALF, D), jnp.float32),
    scratch_types=[
        pltpu.VMEM((NCH, GCH), jnp.int32),
        pltpu.VMEM((2, GCH, D), jnp.float32),
        pltpu.SemaphoreType.DMA,
        pltpu.SemaphoreType.DMA,
        pltpu.SemaphoreType.DMA,
        pltpu.SemaphoreType.DMA,
    ],
)
def _sc_gather(x_hbm, tok_hbm, out_hbm, idx_v, buf_v, g0, g1, o0, o1):
    wid = lax.axis_index("s") * NC + lax.axis_index("c")
    base = wid * TOK_PER_W
    gsem = (g0, g1)
    osem = (o0, o1)

    pltpu.sync_copy(x_hbm.at[wid], idx_v)

    gath = [None, None]
    outc = [None, None]
    for c in range(NCH):
        slot = c & 1
        if outc[slot] is not None:
            outc[slot].wait()
        gath[slot] = pltpu.async_copy(
            tok_hbm.at[idx_v.at[c]], buf_v.at[slot], gsem[slot])
        prev = slot ^ 1
        if gath[prev] is not None:
            gath[prev].wait()
            obase = pl.multiple_of(base + (c - 1) * GCH, GCH)
            outc[prev] = pltpu.async_copy(
                buf_v.at[prev], out_hbm.at[pl.ds(obase, GCH)], osem[prev])
    last = (NCH - 1) & 1
    gath[last].wait()
    obase = pl.multiple_of(base + (NCH - 1) * GCH, GCH)
    outc[last] = pltpu.async_copy(
        buf_v.at[last], out_hbm.at[pl.ds(obase, GCH)], osem[last])
    outc[last ^ 1].wait()
    outc[last].wait()


RS = 256
N_BLK = S // RS
N_BLK_HALF = N_BLK // 2


def _tc_ln_body(h0_ref, h1_ref, pos_ref, g_ref, b_ref, o_ref):
    first = pl.program_id(0) < N_BLK_HALF
    h = jnp.where(first, h0_ref[...], h1_ref[...]) + pos_ref[...]
    mean = jnp.mean(h, axis=-1, keepdims=True)
    cent = h - mean
    var = jnp.mean(cent * cent, axis=-1, keepdims=True)
    o_ref[...] = cent * lax.rsqrt(var + 1e-5) * g_ref[...] + b_ref[...]


_tc_ln = pl.pallas_call(
    _tc_ln_body,
    grid=(N_BLK,),
    in_specs=[
        pl.BlockSpec((RS, B, D), lambda i: (jnp.minimum(i, N_BLK_HALF - 1), 0, 0)),
        pl.BlockSpec((RS, B, D), lambda i: (jnp.maximum(i - N_BLK_HALF, 0), 0, 0)),
        pl.BlockSpec((RS, 1, D), lambda i: (i, 0, 0)),
        pl.BlockSpec((1, 1, D), lambda i: (0, 0, 0)),
        pl.BlockSpec((1, 1, D), lambda i: (0, 0, 0)),
    ],
    out_specs=pl.BlockSpec((RS, B, D), lambda i: (i, 0, 0)),
    out_shape=jax.ShapeDtypeStruct((S, B, D), jnp.float32),
    compiler_params=pltpu.CompilerParams(
        dimension_semantics=("arbitrary",),
    ),
)


def kernel(x, tok_table, pos_table, gamma, beta):
    xf = x.reshape(-1).astype(jnp.int32)
    rows0 = _sc_gather(xf[:HALF].reshape(NW, NCH, GCH), tok_table)
    rows1 = _sc_gather(xf[HALF:].reshape(NW, NCH, GCH), tok_table)
    return _tc_ln(
        rows0.reshape(S_HALF, B, D),
        rows1.reshape(S_HALF, B, D),
        pos_table.reshape(S, 1, D),
        gamma.reshape(1, 1, D),
        beta.reshape(1, 1, D),
    )
